# baseline (speedup 1.0000x reference)
import functools

import jax
import jax.numpy as jnp
from jax import lax
from jax.experimental import pallas as pl
from jax.experimental.pallas import tpu as pltpu
from jax.experimental.pallas import tpu_sc as plsc

N = 10000
E = 320000
VOCAB = 512
BAG = 16
HID = 128
OUT = 7

NW = 32
N_PAD = 10240
ROWS_W = N_PAD // NW
GRP = 128
EGRP = 32
CHG = 16
NCH = 20
NG_E = NCH * CHG
EPW = NG_E * EGRP
E_PAD = NW * EPW
NG_B = (ROWS_W * BAG) // GRP

_MESH = plsc.VectorSubcoreMesh(core_axis_name="c", subcore_axis_name="s")


def _emb_body(x_ref, emb_ref, w_ref, o_ref):
    blk = x_ref.shape[0]
    cnt = jnp.zeros((blk, VOCAB), jnp.float32)
    iota = lax.broadcasted_iota(jnp.int32, (blk, VOCAB), 1)
    for b in range(BAG):
        cnt = cnt + (x_ref[:, b][:, None] == iota).astype(jnp.float32)
    h = jnp.maximum(
        jnp.dot(cnt, emb_ref[...], preferred_element_type=jnp.float32)
        * (1.0 / BAG), 0.0)
    o_ref[...] = jnp.dot(h, w_ref[...], preferred_element_type=jnp.float32)


def _emb_call(x, emb, w0):
    blk = N_PAD // 8
    return pl.pallas_call(
        _emb_body,
        grid=(8,),
        in_specs=[
            pl.BlockSpec((blk, BAG), lambda i: (i, 0)),
            pl.BlockSpec((VOCAB, HID), lambda i: (0, 0)),
            pl.BlockSpec((HID, HID), lambda i: (0, 0)),
        ],
        out_specs=pl.BlockSpec((blk, HID), lambda i: (i, 0)),
        out_shape=jax.ShapeDtypeStruct((N_PAD, HID), jnp.float32),
    )(x, emb, w0)


def _gcn_body(g_hbm, srcr_hbm, dstr_hbm, out_hbm, src_v, dst_v, rows_v,
              acc_sh, sem):
    c = lax.axis_index("c")
    s = lax.axis_index("s")
    wid = s * 2 + c
    rows_per_tile = N_PAD // 16

    for r in range(2 * EGRP):
        for cg in range(HID // 16):
            rows_v[r, pl.ds(cg * 16, 16)] = jnp.zeros((16,), jnp.float32)
    for i in range(rows_per_tile // (2 * EGRP)):
        pltpu.sync_copy(
            rows_v.at[pl.ds(0, 2 * EGRP)],
            acc_sh.at[pl.ds(s * rows_per_tile + i * 2 * EGRP, 2 * EGRP)])
    plsc.subcore_barrier()

    pltpu.sync_copy(srcr_hbm.at[wid, pl.ds(0, CHG)], src_v.at[0])
    pltpu.sync_copy(dstr_hbm.at[wid, pl.ds(0, CHG)], dst_v.at[0])

    def _gather(g):
        slot = (g // CHG) % 2
        pltpu.async_copy(g_hbm.at[src_v.at[slot, g % CHG]],
                         rows_v.at[pl.ds((g % 8) * EGRP, EGRP)], sem)

    for _pg in range(7):
        _gather(_pg)

    def body(g, carry):
        @pl.when(jnp.logical_and(g % CHG == 0, g + CHG < NG_E))
        def _():
            nslot = ((g // CHG) + 1) % 2
            pltpu.sync_copy(
                srcr_hbm.at[wid, pl.ds((g // CHG + 1) * CHG, CHG)],
                src_v.at[nslot])
            pltpu.sync_copy(
                dstr_hbm.at[wid, pl.ds((g // CHG + 1) * CHG, CHG)],
                dst_v.at[nslot])

        slot = (g // CHG) % 2
        buf = (g % 8) * EGRP
        pltpu.make_async_copy(
            g_hbm.at[src_v.at[slot, g % CHG]], rows_v.at[pl.ds(buf, EGRP)],
            sem).wait()

        @pl.when(g + 7 < NG_E)
        def _():
            _gather(g + 7)

        pltpu.sync_copy(rows_v.at[pl.ds(buf, EGRP)],
                        acc_sh.at[dst_v.at[slot, g % CHG]], add=True)
        return carry

    lax.fori_loop(0, NG_E, body, 0)
    plsc.subcore_barrier()
    pltpu.sync_copy(acc_sh.at[pl.ds(s * rows_per_tile, rows_per_tile)],
                    out_hbm.at[c, pl.ds(s * rows_per_tile, rows_per_tile)])


_gcn_call = pl.kernel(
    _gcn_body,
    out_type=jax.ShapeDtypeStruct((2, N_PAD, HID), jnp.float32),
    mesh=_MESH,
    scratch_types=[
        pltpu.VMEM((2, CHG, EGRP), jnp.int32),
        pltpu.VMEM((2, CHG, EGRP), jnp.int32),
        pltpu.VMEM((8 * EGRP, HID), jnp.float32),
        pltpu.VMEM_SHARED((N_PAD, HID), jnp.float32),
        pltpu.SemaphoreType.DMA,
    ],
)


def _mm_body(a_ref, w_ref, o_ref):
    o_ref[...] = jnp.dot(a_ref[...], w_ref[...],
                         preferred_element_type=jnp.float32)


def _mm_fused_body(p0_ref, p1_ref, w_ref, o_ref):
    h = jnp.maximum(p0_ref[0] + p1_ref[0], 0.0)
    o_ref[...] = jnp.dot(h, w_ref[...], preferred_element_type=jnp.float32)


_MM_BLK = N_PAD // 4


def _mm(h, w):
    return pl.pallas_call(
        _mm_body,
        grid=(4,),
        in_specs=[
            pl.BlockSpec((_MM_BLK, HID), lambda i: (i, 0)),
            pl.BlockSpec((HID, HID), lambda i: (0, 0)),
        ],
        out_specs=pl.BlockSpec((_MM_BLK, HID), lambda i: (i, 0)),
        out_shape=jax.ShapeDtypeStruct((N_PAD, HID), jnp.float32),
    )(h, w)


def _mm_fused(p, w):
    return pl.pallas_call(
        _mm_fused_body,
        grid=(4,),
        in_specs=[
            pl.BlockSpec((1, _MM_BLK, HID), lambda i: (0, i, 0)),
            pl.BlockSpec((1, _MM_BLK, HID), lambda i: (1, i, 0)),
            pl.BlockSpec((HID, HID), lambda i: (0, 0)),
        ],
        out_specs=pl.BlockSpec((_MM_BLK, HID), lambda i: (i, 0)),
        out_shape=jax.ShapeDtypeStruct((N_PAD, HID), jnp.float32),
    )(p, p, w)


def kernel(x, edge_index, emb, W0, W1, W2, W3, W4, lin_w):
    xp = jnp.zeros((N_PAD, BAG), jnp.int32).at[:N].set(x)

    src = edge_index[0]
    dst = edge_index[1]
    pad_ids = (jnp.arange(E_PAD - E, dtype=jnp.int32) % (N_PAD - N)) + N
    srcp = jnp.concatenate([src, pad_ids]).reshape(NW, NG_E, EGRP)
    dstp = jnp.concatenate([dst, pad_ids]).reshape(NW, NG_E, EGRP)

    g = _emb_call(xp, emb, W0)
    for W in (W1, W2, W3, W4):
        p = _gcn_call(g, srcp, dstp)
        g = _mm_fused(p, W)
    p = _gcn_call(g, srcp, dstp)

    lin_pad = jnp.zeros((HID, 128), jnp.float32).at[:, :OUT].set(lin_w)
    out = _mm_fused(p, lin_pad)
    return out[:N, :OUT]


# speedup vs baseline: 1.0428x; 1.0428x over previous
import functools

import jax
import jax.numpy as jnp
from jax import lax
from jax.experimental import pallas as pl
from jax.experimental.pallas import tpu as pltpu
from jax.experimental.pallas import tpu_sc as plsc

N = 10000
E = 320000
VOCAB = 512
BAG = 16
HID = 128
OUT = 7

NW = 32
N_PAD = 10240
ROWS_W = N_PAD // NW
GRP = 128
EGRP = 64
CHG = 8
NCH = 20
NG_E = NCH * CHG
EPW = NG_E * EGRP
E_PAD = NW * EPW
NG_B = (ROWS_W * BAG) // GRP

_MESH = plsc.VectorSubcoreMesh(core_axis_name="c", subcore_axis_name="s")


def _emb_body(x_ref, emb_ref, w_ref, o_ref):
    blk = x_ref.shape[0]
    cnt = jnp.zeros((blk, VOCAB), jnp.float32)
    iota = lax.broadcasted_iota(jnp.int32, (blk, VOCAB), 1)
    for b in range(BAG):
        cnt = cnt + (x_ref[:, b][:, None] == iota).astype(jnp.float32)
    h = jnp.maximum(
        jnp.dot(cnt, emb_ref[...], preferred_element_type=jnp.float32)
        * (1.0 / BAG), 0.0)
    o_ref[...] = jnp.dot(h, w_ref[...], preferred_element_type=jnp.float32)


def _emb_call(x, emb, w0):
    blk = N_PAD // 8
    return pl.pallas_call(
        _emb_body,
        grid=(8,),
        in_specs=[
            pl.BlockSpec((blk, BAG), lambda i: (i, 0)),
            pl.BlockSpec((VOCAB, HID), lambda i: (0, 0)),
            pl.BlockSpec((HID, HID), lambda i: (0, 0)),
        ],
        out_specs=pl.BlockSpec((blk, HID), lambda i: (i, 0)),
        out_shape=jax.ShapeDtypeStruct((N_PAD, HID), jnp.float32),
    )(x, emb, w0)


def _gcn_body(g_hbm, srcr_hbm, dstr_hbm, out_hbm, src_v, dst_v, rows_v,
              acc_sh, sem):
    c = lax.axis_index("c")
    s = lax.axis_index("s")
    wid = s * 2 + c
    rows_per_tile = N_PAD // 16

    for r in range(2 * EGRP):
        for cg in range(HID // 16):
            rows_v[r, pl.ds(cg * 16, 16)] = jnp.zeros((16,), jnp.float32)
    for i in range(rows_per_tile // (2 * EGRP)):
        pltpu.sync_copy(
            rows_v.at[pl.ds(0, 2 * EGRP)],
            acc_sh.at[pl.ds(s * rows_per_tile + i * 2 * EGRP, 2 * EGRP)])
    plsc.subcore_barrier()

    pltpu.sync_copy(srcr_hbm.at[wid, pl.ds(0, CHG)], src_v.at[0])
    pltpu.sync_copy(dstr_hbm.at[wid, pl.ds(0, CHG)], dst_v.at[0])

    def _gather(g):
        slot = (g // CHG) % 2
        pltpu.async_copy(g_hbm.at[src_v.at[slot, g % CHG]],
                         rows_v.at[pl.ds((g % 4) * EGRP, EGRP)], sem)

    for _pg in range(3):
        _gather(_pg)

    def body(g, carry):
        @pl.when(jnp.logical_and(g % CHG == 0, g + CHG < NG_E))
        def _():
            nslot = ((g // CHG) + 1) % 2
            pltpu.sync_copy(
                srcr_hbm.at[wid, pl.ds((g // CHG + 1) * CHG, CHG)],
                src_v.at[nslot])
            pltpu.sync_copy(
                dstr_hbm.at[wid, pl.ds((g // CHG + 1) * CHG, CHG)],
                dst_v.at[nslot])

        slot = (g // CHG) % 2
        buf = (g % 4) * EGRP
        pltpu.make_async_copy(
            g_hbm.at[src_v.at[slot, g % CHG]], rows_v.at[pl.ds(buf, EGRP)],
            sem).wait()

        @pl.when(g + 3 < NG_E)
        def _():
            _gather(g + 3)

        pltpu.sync_copy(rows_v.at[pl.ds(buf, EGRP)],
                        acc_sh.at[dst_v.at[slot, g % CHG]], add=True)
        return carry

    lax.fori_loop(0, NG_E, body, 0)
    plsc.subcore_barrier()
    pltpu.sync_copy(acc_sh.at[pl.ds(s * rows_per_tile, rows_per_tile)],
                    out_hbm.at[c, pl.ds(s * rows_per_tile, rows_per_tile)])


_gcn_call = pl.kernel(
    _gcn_body,
    out_type=jax.ShapeDtypeStruct((2, N_PAD, HID), jnp.float32),
    mesh=_MESH,
    scratch_types=[
        pltpu.VMEM((2, CHG, EGRP), jnp.int32),
        pltpu.VMEM((2, CHG, EGRP), jnp.int32),
        pltpu.VMEM((4 * EGRP, HID), jnp.float32),
        pltpu.VMEM_SHARED((N_PAD, HID), jnp.float32),
        pltpu.SemaphoreType.DMA,
    ],
)


def _mm_body(a_ref, w_ref, o_ref):
    o_ref[...] = jnp.dot(a_ref[...], w_ref[...],
                         preferred_element_type=jnp.float32)


def _mm_fused_body(p0_ref, p1_ref, w_ref, o_ref):
    h = jnp.maximum(p0_ref[0] + p1_ref[0], 0.0)
    o_ref[...] = jnp.dot(h, w_ref[...], preferred_element_type=jnp.float32)


_MM_BLK = N_PAD // 4


def _mm(h, w):
    return pl.pallas_call(
        _mm_body,
        grid=(4,),
        in_specs=[
            pl.BlockSpec((_MM_BLK, HID), lambda i: (i, 0)),
            pl.BlockSpec((HID, HID), lambda i: (0, 0)),
        ],
        out_specs=pl.BlockSpec((_MM_BLK, HID), lambda i: (i, 0)),
        out_shape=jax.ShapeDtypeStruct((N_PAD, HID), jnp.float32),
    )(h, w)


def _mm_fused(p, w):
    return pl.pallas_call(
        _mm_fused_body,
        grid=(4,),
        in_specs=[
            pl.BlockSpec((1, _MM_BLK, HID), lambda i: (0, i, 0)),
            pl.BlockSpec((1, _MM_BLK, HID), lambda i: (1, i, 0)),
            pl.BlockSpec((HID, HID), lambda i: (0, 0)),
        ],
        out_specs=pl.BlockSpec((_MM_BLK, HID), lambda i: (i, 0)),
        out_shape=jax.ShapeDtypeStruct((N_PAD, HID), jnp.float32),
    )(p, p, w)


def kernel(x, edge_index, emb, W0, W1, W2, W3, W4, lin_w):
    xp = jnp.zeros((N_PAD, BAG), jnp.int32).at[:N].set(x)

    src = edge_index[0]
    dst = edge_index[1]
    pad_ids = (jnp.arange(E_PAD - E, dtype=jnp.int32) % (N_PAD - N)) + N
    srcp = jnp.concatenate([src, pad_ids]).reshape(NW, NG_E, EGRP)
    dstp = jnp.concatenate([dst, pad_ids]).reshape(NW, NG_E, EGRP)

    g = _emb_call(xp, emb, W0)
    for W in (W1, W2, W3, W4):
        p = _gcn_call(g, srcp, dstp)
        g = _mm_fused(p, W)
    p = _gcn_call(g, srcp, dstp)

    lin_pad = jnp.zeros((HID, 128), jnp.float32).at[:, :OUT].set(lin_w)
    out = _mm_fused(p, lin_pad)
    return out[:N, :OUT]


# speedup vs baseline: 1.1057x; 1.0603x over previous
import functools

import jax
import jax.numpy as jnp
from jax import lax
from jax.experimental import pallas as pl
from jax.experimental.pallas import tpu as pltpu
from jax.experimental.pallas import tpu_sc as plsc

N = 10000
E = 320000
VOCAB = 512
BAG = 16
HID = 128
OUT = 7

NW = 32
N_PAD = 10240
ROWS_W = N_PAD // NW
GRP = 128
EGRP = 64
CHG = 16
NCH = 10
NG_E = NCH * CHG
EPW = NG_E * EGRP
E_PAD = NW * EPW
NG_B = (ROWS_W * BAG) // GRP

_MESH = plsc.VectorSubcoreMesh(core_axis_name="c", subcore_axis_name="s")


def _emb_body(x_ref, emb_ref, w_ref, o_ref):
    blk = x_ref.shape[0]
    cnt = jnp.zeros((blk, VOCAB), jnp.float32)
    iota = lax.broadcasted_iota(jnp.int32, (blk, VOCAB), 1)
    for b in range(BAG):
        cnt = cnt + (x_ref[:, b][:, None] == iota).astype(jnp.float32)
    h = jnp.maximum(
        jnp.dot(cnt, emb_ref[...], preferred_element_type=jnp.float32)
        * (1.0 / BAG), 0.0)
    o_ref[...] = jnp.dot(h, w_ref[...], preferred_element_type=jnp.float32)


def _emb_call(x, emb, w0):
    blk = N_PAD // 8
    return pl.pallas_call(
        _emb_body,
        grid=(8,),
        in_specs=[
            pl.BlockSpec((blk, BAG), lambda i: (i, 0)),
            pl.BlockSpec((VOCAB, HID), lambda i: (0, 0)),
            pl.BlockSpec((HID, HID), lambda i: (0, 0)),
        ],
        out_specs=pl.BlockSpec((blk, HID), lambda i: (i, 0)),
        out_shape=jax.ShapeDtypeStruct((N_PAD, HID), jnp.float32),
    )(x, emb, w0)


def _gcn_body(g_hbm, srcr_hbm, dstr_hbm, out_hbm, src_v, dst_v, rows_v,
              acc_sh, sem):
    c = lax.axis_index("c")
    s = lax.axis_index("s")
    wid = s * 2 + c
    rows_per_tile = N_PAD // 16

    for r in range(2 * EGRP):
        for cg in range(HID // 16):
            rows_v[r, pl.ds(cg * 16, 16)] = jnp.zeros((16,), jnp.float32)
    for i in range(rows_per_tile // (2 * EGRP)):
        pltpu.sync_copy(
            rows_v.at[pl.ds(0, 2 * EGRP)],
            acc_sh.at[pl.ds(s * rows_per_tile + i * 2 * EGRP, 2 * EGRP)])
    plsc.subcore_barrier()

    pltpu.sync_copy(srcr_hbm.at[wid, pl.ds(0, CHG)], src_v.at[0])
    pltpu.sync_copy(dstr_hbm.at[wid, pl.ds(0, CHG)], dst_v.at[0])

    def _gather(g):
        slot = (g // CHG) % 2
        pltpu.async_copy(g_hbm.at[src_v.at[slot, g % CHG]],
                         rows_v.at[pl.ds((g % 5) * EGRP, EGRP)], sem)

    for _pg in range(4):
        _gather(_pg)

    def body(g, carry):
        @pl.when(jnp.logical_and(g % CHG == 0, g + CHG < NG_E))
        def _():
            nslot = ((g // CHG) + 1) % 2
            pltpu.sync_copy(
                srcr_hbm.at[wid, pl.ds((g // CHG + 1) * CHG, CHG)],
                src_v.at[nslot])
            pltpu.sync_copy(
                dstr_hbm.at[wid, pl.ds((g // CHG + 1) * CHG, CHG)],
                dst_v.at[nslot])

        slot = (g // CHG) % 2
        buf = (g % 5) * EGRP
        pltpu.make_async_copy(
            g_hbm.at[src_v.at[slot, g % CHG]], rows_v.at[pl.ds(buf, EGRP)],
            sem).wait()

        @pl.when(g + 4 < NG_E)
        def _():
            _gather(g + 4)

        pltpu.sync_copy(rows_v.at[pl.ds(buf, EGRP)],
                        acc_sh.at[dst_v.at[slot, g % CHG]], add=True)
        return carry

    lax.fori_loop(0, NG_E, body, 0)
    plsc.subcore_barrier()
    pltpu.sync_copy(acc_sh.at[pl.ds(s * rows_per_tile, rows_per_tile)],
                    out_hbm.at[c, pl.ds(s * rows_per_tile, rows_per_tile)])


_gcn_call = pl.kernel(
    _gcn_body,
    out_type=jax.ShapeDtypeStruct((2, N_PAD, HID), jnp.float32),
    mesh=_MESH,
    scratch_types=[
        pltpu.VMEM((2, CHG, EGRP), jnp.int32),
        pltpu.VMEM((2, CHG, EGRP), jnp.int32),
        pltpu.VMEM((5 * EGRP, HID), jnp.float32),
        pltpu.VMEM_SHARED((N_PAD, HID), jnp.float32),
        pltpu.SemaphoreType.DMA,
    ],
)


def _mm_body(a_ref, w_ref, o_ref):
    o_ref[...] = jnp.dot(a_ref[...], w_ref[...],
                         preferred_element_type=jnp.float32)


def _mm_fused_body(p0_ref, p1_ref, w_ref, o_ref):
    h = jnp.maximum(p0_ref[0] + p1_ref[0], 0.0)
    o_ref[...] = jnp.dot(h, w_ref[...], preferred_element_type=jnp.float32)


_MM_BLK = N_PAD // 4


def _mm(h, w):
    return pl.pallas_call(
        _mm_body,
        grid=(4,),
        in_specs=[
            pl.BlockSpec((_MM_BLK, HID), lambda i: (i, 0)),
            pl.BlockSpec((HID, HID), lambda i: (0, 0)),
        ],
        out_specs=pl.BlockSpec((_MM_BLK, HID), lambda i: (i, 0)),
        out_shape=jax.ShapeDtypeStruct((N_PAD, HID), jnp.float32),
    )(h, w)


def _mm_fused(p, w):
    return pl.pallas_call(
        _mm_fused_body,
        grid=(4,),
        in_specs=[
            pl.BlockSpec((1, _MM_BLK, HID), lambda i: (0, i, 0)),
            pl.BlockSpec((1, _MM_BLK, HID), lambda i: (1, i, 0)),
            pl.BlockSpec((HID, HID), lambda i: (0, 0)),
        ],
        out_specs=pl.BlockSpec((_MM_BLK, HID), lambda i: (i, 0)),
        out_shape=jax.ShapeDtypeStruct((N_PAD, HID), jnp.float32),
    )(p, p, w)


def kernel(x, edge_index, emb, W0, W1, W2, W3, W4, lin_w):
    xp = jnp.zeros((N_PAD, BAG), jnp.int32).at[:N].set(x)

    src = edge_index[0]
    dst = edge_index[1]
    pad_ids = (jnp.arange(E_PAD - E, dtype=jnp.int32) % (N_PAD - N)) + N
    srcp = jnp.concatenate([src, pad_ids]).reshape(NW, NG_E, EGRP)
    dstp = jnp.concatenate([dst, pad_ids]).reshape(NW, NG_E, EGRP)

    g = _emb_call(xp, emb, W0)
    for W in (W1, W2, W3, W4):
        p = _gcn_call(g, srcp, dstp)
        g = _mm_fused(p, W)
    p = _gcn_call(g, srcp, dstp)

    lin_pad = jnp.zeros((HID, 128), jnp.float32).at[:, :OUT].set(lin_w)
    out = _mm_fused(p, lin_pad)
    return out[:N, :OUT]


# speedup vs baseline: 1.1871x; 1.0737x over previous
import functools

import jax
import jax.numpy as jnp
from jax import lax
from jax.experimental import pallas as pl
from jax.experimental.pallas import tpu as pltpu
from jax.experimental.pallas import tpu_sc as plsc

N = 10000
E = 320000
VOCAB = 512
BAG = 16
HID = 128
OUT = 7

NW = 32
N_PAD = 10240
ROWS_W = N_PAD // NW
GRP = 128
EGRP = 64
CHG = 16
NCH = 10
NG_E = NCH * CHG
EPW = NG_E * EGRP
E_PAD = NW * EPW
NG_B = (ROWS_W * BAG) // GRP

_MESH = plsc.VectorSubcoreMesh(core_axis_name="c", subcore_axis_name="s")


def _emb_body(x_ref, emb_ref, w_ref, o_ref):
    blk = x_ref.shape[0]
    cnt = jnp.zeros((blk, VOCAB), jnp.float32)
    iota = lax.broadcasted_iota(jnp.int32, (blk, VOCAB), 1)
    for b in range(BAG):
        cnt = cnt + (x_ref[:, b][:, None] == iota).astype(jnp.float32)
    h = jnp.maximum(
        jnp.dot(cnt, emb_ref[...], preferred_element_type=jnp.float32)
        * (1.0 / BAG), 0.0)
    o_ref[...] = jnp.dot(h, w_ref[...], preferred_element_type=jnp.float32)


def _emb_call(x, emb, w0):
    blk = N_PAD // 8
    return pl.pallas_call(
        _emb_body,
        grid=(8,),
        in_specs=[
            pl.BlockSpec((blk, BAG), lambda i: (i, 0)),
            pl.BlockSpec((VOCAB, HID), lambda i: (0, 0)),
            pl.BlockSpec((HID, HID), lambda i: (0, 0)),
        ],
        out_specs=pl.BlockSpec((blk, HID), lambda i: (i, 0)),
        out_shape=jax.ShapeDtypeStruct((N_PAD, HID), jnp.float32),
    )(x, emb, w0)


def _gcn_body(g_hbm, srcr_hbm, dstr_hbm, out_hbm, src_v, dst_v, rows_v,
              acc_sh, sem, sem_i, sem_s0, sem_s1, sem_s2, sem_s3, sem_s4):
    c = lax.axis_index("c")
    s = lax.axis_index("s")
    wid = s * 2 + c
    rows_per_tile = N_PAD // 16

    for r in range(2 * EGRP):
        for cg in range(HID // 16):
            rows_v[r, pl.ds(cg * 16, 16)] = jnp.zeros((16,), jnp.float32)
    for i in range(rows_per_tile // (2 * EGRP)):
        pltpu.sync_copy(
            rows_v.at[pl.ds(0, 2 * EGRP)],
            acc_sh.at[pl.ds(s * rows_per_tile + i * 2 * EGRP, 2 * EGRP)])
    plsc.subcore_barrier()

    pltpu.sync_copy(srcr_hbm.at[wid, pl.ds(0, CHG)], src_v.at[0])
    pltpu.sync_copy(dstr_hbm.at[wid, pl.ds(0, CHG)], dst_v.at[0])

    def _gather(g):
        slot = (g // CHG) % 2
        pltpu.async_copy(g_hbm.at[src_v.at[slot, g % CHG]],
                         rows_v.at[pl.ds((g % 5) * EGRP, EGRP)], sem)

    for _pg in range(4):
        _gather(_pg)

    def _scat_start(g, k, sem_k):
        slot = (g // CHG) % 2
        pltpu.async_copy(rows_v.at[pl.ds(k * EGRP, EGRP)],
                         acc_sh.at[dst_v.at[slot, g % CHG]], sem_k, add=True)

    def _scat_wait(g, k, sem_k):
        slot = (g // CHG) % 2
        pltpu.make_async_copy(rows_v.at[pl.ds(k * EGRP, EGRP)],
                              acc_sh.at[dst_v.at[slot, g % CHG]], sem_k).wait()

    def _step(g, k, sems):
        @pl.when(g > 0)
        def _():
            _scat_wait(g - 1, (k + 4) % 5, sems[(k + 4) % 5])

        @pl.when(jnp.logical_and(g % CHG == 0, g + CHG < NG_E))
        def _():
            nslot = ((g // CHG) + 1) % 2
            pltpu.async_copy(
                srcr_hbm.at[wid, pl.ds((g // CHG + 1) * CHG, CHG)],
                src_v.at[nslot], sem_i)
            pltpu.async_copy(
                dstr_hbm.at[wid, pl.ds((g // CHG + 1) * CHG, CHG)],
                dst_v.at[nslot], sem_i)

        @pl.when(jnp.logical_and(g % CHG == CHG - 4, g + 4 < NG_E))
        def _():
            nslot = ((g // CHG) + 1) % 2
            pltpu.make_async_copy(
                srcr_hbm.at[wid, pl.ds((g // CHG + 1) * CHG, CHG)],
                src_v.at[nslot], sem_i).wait()
            pltpu.make_async_copy(
                dstr_hbm.at[wid, pl.ds((g // CHG + 1) * CHG, CHG)],
                dst_v.at[nslot], sem_i).wait()

        slot = (g // CHG) % 2
        pltpu.make_async_copy(
            g_hbm.at[src_v.at[slot, g % CHG]],
            rows_v.at[pl.ds(k * EGRP, EGRP)], sem).wait()

        @pl.when(g + 4 < NG_E)
        def _():
            _gather(g + 4)

        _scat_start(g, k, sems[k])

    def body(i, carry):
        sems = (sem_s0, sem_s1, sem_s2, sem_s3, sem_s4)
        for k in range(5):
            _step(5 * i + k, k, sems)
        return carry

    lax.fori_loop(0, NG_E // 5, body, 0)
    _scat_wait(NG_E - 1, (NG_E - 1) % 5, sem_s4)
    plsc.subcore_barrier()
    pltpu.sync_copy(acc_sh.at[pl.ds(s * rows_per_tile, rows_per_tile)],
                    out_hbm.at[c, pl.ds(s * rows_per_tile, rows_per_tile)])


_gcn_call = pl.kernel(
    _gcn_body,
    out_type=jax.ShapeDtypeStruct((2, N_PAD, HID), jnp.float32),
    mesh=_MESH,
    scratch_types=[
        pltpu.VMEM((2, CHG, EGRP), jnp.int32),
        pltpu.VMEM((2, CHG, EGRP), jnp.int32),
        pltpu.VMEM((5 * EGRP, HID), jnp.float32),
        pltpu.VMEM_SHARED((N_PAD, HID), jnp.float32),
        pltpu.SemaphoreType.DMA,
        pltpu.SemaphoreType.DMA,
        pltpu.SemaphoreType.DMA,
        pltpu.SemaphoreType.DMA,
        pltpu.SemaphoreType.DMA,
        pltpu.SemaphoreType.DMA,
        pltpu.SemaphoreType.DMA,
    ],
)


def _mm_body(a_ref, w_ref, o_ref):
    o_ref[...] = jnp.dot(a_ref[...], w_ref[...],
                         preferred_element_type=jnp.float32)


def _mm_fused_body(p0_ref, p1_ref, w_ref, o_ref):
    h = jnp.maximum(p0_ref[0] + p1_ref[0], 0.0)
    o_ref[...] = jnp.dot(h, w_ref[...], preferred_element_type=jnp.float32)


_MM_BLK = N_PAD // 4


def _mm(h, w):
    return pl.pallas_call(
        _mm_body,
        grid=(4,),
        in_specs=[
            pl.BlockSpec((_MM_BLK, HID), lambda i: (i, 0)),
            pl.BlockSpec((HID, HID), lambda i: (0, 0)),
        ],
        out_specs=pl.BlockSpec((_MM_BLK, HID), lambda i: (i, 0)),
        out_shape=jax.ShapeDtypeStruct((N_PAD, HID), jnp.float32),
    )(h, w)


def _mm_fused(p, w):
    return pl.pallas_call(
        _mm_fused_body,
        grid=(4,),
        in_specs=[
            pl.BlockSpec((1, _MM_BLK, HID), lambda i: (0, i, 0)),
            pl.BlockSpec((1, _MM_BLK, HID), lambda i: (1, i, 0)),
            pl.BlockSpec((HID, HID), lambda i: (0, 0)),
        ],
        out_specs=pl.BlockSpec((_MM_BLK, HID), lambda i: (i, 0)),
        out_shape=jax.ShapeDtypeStruct((N_PAD, HID), jnp.float32),
    )(p, p, w)


def kernel(x, edge_index, emb, W0, W1, W2, W3, W4, lin_w):
    xp = jnp.zeros((N_PAD, BAG), jnp.int32).at[:N].set(x)

    src = edge_index[0]
    dst = edge_index[1]
    pad_ids = (jnp.arange(E_PAD - E, dtype=jnp.int32) % (N_PAD - N)) + N
    srcp = jnp.concatenate([src, pad_ids]).reshape(NW, NG_E, EGRP)
    dstp = jnp.concatenate([dst, pad_ids]).reshape(NW, NG_E, EGRP)

    g = _emb_call(xp, emb, W0)
    for W in (W1, W2, W3, W4):
        p = _gcn_call(g, srcp, dstp)
        g = _mm_fused(p, W)
    p = _gcn_call(g, srcp, dstp)

    lin_pad = jnp.zeros((HID, 128), jnp.float32).at[:, :OUT].set(lin_w)
    out = _mm_fused(p, lin_pad)
    return out[:N, :OUT]


# speedup vs baseline: 1.2100x; 1.0193x over previous
import functools

import jax
import jax.numpy as jnp
from jax import lax
from jax.experimental import pallas as pl
from jax.experimental.pallas import tpu as pltpu
from jax.experimental.pallas import tpu_sc as plsc

N = 10000
E = 320000
VOCAB = 512
BAG = 16
HID = 128
OUT = 7

NW = 32
N_PAD = 10240
ROWS_W = N_PAD // NW
GRP = 128
EGRP = 64
CHG = 16
NCH = 10
NG_E = NCH * CHG
EPW = NG_E * EGRP
E_PAD = NW * EPW
NG_B = (ROWS_W * BAG) // GRP

_MESH = plsc.VectorSubcoreMesh(core_axis_name="c", subcore_axis_name="s")


def _emb_body(x_ref, emb_ref, w_ref, o_ref):
    blk = x_ref.shape[0]
    cnt = jnp.zeros((blk, VOCAB), jnp.float32)
    iota = lax.broadcasted_iota(jnp.int32, (blk, VOCAB), 1)
    for b in range(BAG):
        cnt = cnt + (x_ref[:, b][:, None] == iota).astype(jnp.float32)
    h = jnp.maximum(
        jnp.dot(cnt, emb_ref[...], preferred_element_type=jnp.float32)
        * (1.0 / BAG), 0.0)
    o_ref[...] = jnp.dot(h, w_ref[...], preferred_element_type=jnp.float32)


def _emb_call(x, emb, w0):
    blk = N_PAD // 8
    return pl.pallas_call(
        _emb_body,
        grid=(8,),
        in_specs=[
            pl.BlockSpec((blk, BAG), lambda i: (i, 0)),
            pl.BlockSpec((VOCAB, HID), lambda i: (0, 0)),
            pl.BlockSpec((HID, HID), lambda i: (0, 0)),
        ],
        out_specs=pl.BlockSpec((blk, HID), lambda i: (i, 0)),
        out_shape=jax.ShapeDtypeStruct((N_PAD, HID), jnp.float32),
    )(x, emb, w0)


def _gcn_body(g_hbm, srcr_hbm, dstr_hbm, out_hbm, src_v, dst_v, rows_v,
              acc_sh, sem, sem_i, sem_s0, sem_s1, sem_s2, sem_s3, sem_s4):
    c = lax.axis_index("c")
    s = lax.axis_index("s")
    wid = s * 2 + c
    rows_per_tile = N_PAD // 16

    pltpu.sync_copy(srcr_hbm.at[wid, pl.ds(0, CHG)], src_v.at[0])
    pltpu.sync_copy(dstr_hbm.at[wid, pl.ds(0, CHG)], dst_v.at[0])

    def _gather(g):
        slot = (g // CHG) % 2
        pltpu.async_copy(g_hbm.at[src_v.at[slot, g % CHG]],
                         rows_v.at[pl.ds((g % 5) * EGRP, EGRP)], sem)

    for _pg in range(4):
        _gather(_pg)

    for r in range(EGRP):
        for cg in range(HID // 16):
            rows_v[4 * EGRP + r, pl.ds(cg * 16, 16)] = jnp.zeros(
                (16,), jnp.float32)
    for i in range(rows_per_tile // EGRP):
        pltpu.sync_copy(
            rows_v.at[pl.ds(4 * EGRP, EGRP)],
            acc_sh.at[pl.ds(s * rows_per_tile + i * EGRP, EGRP)])
    plsc.subcore_barrier()

    def _scat_start(g, k, sem_k):
        slot = (g // CHG) % 2
        pltpu.async_copy(rows_v.at[pl.ds(k * EGRP, EGRP)],
                         acc_sh.at[dst_v.at[slot, g % CHG]], sem_k, add=True)

    def _scat_wait(g, k, sem_k):
        slot = (g // CHG) % 2
        pltpu.make_async_copy(rows_v.at[pl.ds(k * EGRP, EGRP)],
                              acc_sh.at[dst_v.at[slot, g % CHG]], sem_k).wait()

    def _step(g, k, sems):
        @pl.when(g > 0)
        def _():
            _scat_wait(g - 1, (k + 4) % 5, sems[(k + 4) % 5])

        @pl.when(jnp.logical_and(g % CHG == 0, g + CHG < NG_E))
        def _():
            nslot = ((g // CHG) + 1) % 2
            pltpu.async_copy(
                srcr_hbm.at[wid, pl.ds((g // CHG + 1) * CHG, CHG)],
                src_v.at[nslot], sem_i)
            pltpu.async_copy(
                dstr_hbm.at[wid, pl.ds((g // CHG + 1) * CHG, CHG)],
                dst_v.at[nslot], sem_i)

        @pl.when(jnp.logical_and(g % CHG == CHG - 4, g + 4 < NG_E))
        def _():
            nslot = ((g // CHG) + 1) % 2
            pltpu.make_async_copy(
                srcr_hbm.at[wid, pl.ds((g // CHG + 1) * CHG, CHG)],
                src_v.at[nslot], sem_i).wait()
            pltpu.make_async_copy(
                dstr_hbm.at[wid, pl.ds((g // CHG + 1) * CHG, CHG)],
                dst_v.at[nslot], sem_i).wait()

        slot = (g // CHG) % 2
        pltpu.make_async_copy(
            g_hbm.at[src_v.at[slot, g % CHG]],
            rows_v.at[pl.ds(k * EGRP, EGRP)], sem).wait()

        @pl.when(g + 4 < NG_E)
        def _():
            _gather(g + 4)

        _scat_start(g, k, sems[k])

    def body(i, carry):
        sems = (sem_s0, sem_s1, sem_s2, sem_s3, sem_s4)
        for k in range(5):
            _step(5 * i + k, k, sems)
        return carry

    lax.fori_loop(0, NG_E // 5, body, 0)
    _scat_wait(NG_E - 1, (NG_E - 1) % 5, sem_s4)
    plsc.subcore_barrier()
    pltpu.sync_copy(acc_sh.at[pl.ds(s * rows_per_tile, rows_per_tile)],
                    out_hbm.at[c, pl.ds(s * rows_per_tile, rows_per_tile)])


_gcn_call = pl.kernel(
    _gcn_body,
    out_type=jax.ShapeDtypeStruct((2, N_PAD, HID), jnp.float32),
    mesh=_MESH,
    scratch_types=[
        pltpu.VMEM((2, CHG, EGRP), jnp.int32),
        pltpu.VMEM((2, CHG, EGRP), jnp.int32),
        pltpu.VMEM((5 * EGRP, HID), jnp.float32),
        pltpu.VMEM_SHARED((N_PAD, HID), jnp.float32),
        pltpu.SemaphoreType.DMA,
        pltpu.SemaphoreType.DMA,
        pltpu.SemaphoreType.DMA,
        pltpu.SemaphoreType.DMA,
        pltpu.SemaphoreType.DMA,
        pltpu.SemaphoreType.DMA,
        pltpu.SemaphoreType.DMA,
    ],
)


def _mm_body(a_ref, w_ref, o_ref):
    o_ref[...] = jnp.dot(a_ref[...], w_ref[...],
                         preferred_element_type=jnp.float32)


def _mm_fused_body(p0_ref, p1_ref, w_ref, o_ref):
    h = jnp.maximum(p0_ref[0] + p1_ref[0], 0.0)
    o_ref[...] = jnp.dot(h, w_ref[...], preferred_element_type=jnp.float32)


_MM_BLK = N_PAD // 4


def _mm(h, w):
    return pl.pallas_call(
        _mm_body,
        grid=(4,),
        in_specs=[
            pl.BlockSpec((_MM_BLK, HID), lambda i: (i, 0)),
            pl.BlockSpec((HID, HID), lambda i: (0, 0)),
        ],
        out_specs=pl.BlockSpec((_MM_BLK, HID), lambda i: (i, 0)),
        out_shape=jax.ShapeDtypeStruct((N_PAD, HID), jnp.float32),
    )(h, w)


def _mm_fused(p, w):
    return pl.pallas_call(
        _mm_fused_body,
        grid=(4,),
        in_specs=[
            pl.BlockSpec((1, _MM_BLK, HID), lambda i: (0, i, 0)),
            pl.BlockSpec((1, _MM_BLK, HID), lambda i: (1, i, 0)),
            pl.BlockSpec((HID, HID), lambda i: (0, 0)),
        ],
        out_specs=pl.BlockSpec((_MM_BLK, HID), lambda i: (i, 0)),
        out_shape=jax.ShapeDtypeStruct((N_PAD, HID), jnp.float32),
    )(p, p, w)


def kernel(x, edge_index, emb, W0, W1, W2, W3, W4, lin_w):
    xp = jnp.zeros((N_PAD, BAG), jnp.int32).at[:N].set(x)

    src = edge_index[0]
    dst = edge_index[1]
    pad_ids = (jnp.arange(E_PAD - E, dtype=jnp.int32) % (N_PAD - N)) + N
    srcp = jnp.concatenate([src, pad_ids]).reshape(NW, NG_E, EGRP)
    dstp = jnp.concatenate([dst, pad_ids]).reshape(NW, NG_E, EGRP)

    g = _emb_call(xp, emb, W0)
    for W in (W1, W2, W3, W4):
        p = _gcn_call(g, srcp, dstp)
        g = _mm_fused(p, W)
    p = _gcn_call(g, srcp, dstp)

    lin_pad = jnp.zeros((HID, 128), jnp.float32).at[:, :OUT].set(lin_w)
    out = _mm_fused(p, lin_pad)
    return out[:N, :OUT]


# speedup vs baseline: 1.2218x; 1.0098x over previous
import functools

import jax
import jax.numpy as jnp
from jax import lax
from jax.experimental import pallas as pl
from jax.experimental.pallas import tpu as pltpu
from jax.experimental.pallas import tpu_sc as plsc

N = 10000
E = 320000
VOCAB = 512
BAG = 16
HID = 128
OUT = 7

NW = 32
N_PAD = 10240
ROWS_W = N_PAD // NW
GRP = 128
EGRP = 64
CHG = 16
NCH = 10
NG_E = NCH * CHG
EPW = NG_E * EGRP
E_PAD = NW * EPW
NG_B = (ROWS_W * BAG) // GRP

_MESH = plsc.VectorSubcoreMesh(core_axis_name="c", subcore_axis_name="s")


def _emb_body(x_ref, emb_ref, w_ref, o_ref):
    blk = x_ref.shape[0]
    cnt = jnp.zeros((blk, VOCAB), jnp.float32)
    iota = lax.broadcasted_iota(jnp.int32, (blk, VOCAB), 1)
    for b in range(BAG):
        cnt = cnt + (x_ref[:, b][:, None] == iota).astype(jnp.float32)
    h = jnp.maximum(
        jnp.dot(cnt, emb_ref[...], preferred_element_type=jnp.float32)
        * (1.0 / BAG), 0.0)
    o_ref[...] = jnp.dot(h, w_ref[...], preferred_element_type=jnp.float32)


def _emb_call(x, emb, w0):
    blk = N_PAD // 4
    return pl.pallas_call(
        _emb_body,
        grid=(4,),
        in_specs=[
            pl.BlockSpec((blk, BAG), lambda i: (i, 0)),
            pl.BlockSpec((VOCAB, HID), lambda i: (0, 0)),
            pl.BlockSpec((HID, HID), lambda i: (0, 0)),
        ],
        out_specs=pl.BlockSpec((blk, HID), lambda i: (i, 0)),
        out_shape=jax.ShapeDtypeStruct((N_PAD, HID), jnp.float32),
    )(x, emb, w0)


def _gcn_body(g_hbm, srcr_hbm, dstr_hbm, out_hbm, src_v, dst_v, rows_v,
              acc_sh, sem, sem_i, sem_s0, sem_s1, sem_s2, sem_s3, sem_s4):
    c = lax.axis_index("c")
    s = lax.axis_index("s")
    wid = s * 2 + c
    rows_per_tile = N_PAD // 16

    pltpu.sync_copy(srcr_hbm.at[wid, pl.ds(0, CHG)], src_v.at[0])
    pltpu.sync_copy(dstr_hbm.at[wid, pl.ds(0, CHG)], dst_v.at[0])

    def _gather(g):
        slot = (g // CHG) % 2
        pltpu.async_copy(g_hbm.at[src_v.at[slot, g % CHG]],
                         rows_v.at[pl.ds((g % 5) * EGRP, EGRP)], sem)

    for _pg in range(4):
        _gather(_pg)

    for r in range(EGRP):
        for cg in range(HID // 16):
            rows_v[4 * EGRP + r, pl.ds(cg * 16, 16)] = jnp.zeros(
                (16,), jnp.float32)
    for i in range(rows_per_tile // EGRP):
        pltpu.sync_copy(
            rows_v.at[pl.ds(4 * EGRP, EGRP)],
            acc_sh.at[pl.ds(s * rows_per_tile + i * EGRP, EGRP)])
    plsc.subcore_barrier()

    def _scat_start(g, k, sem_k):
        slot = (g // CHG) % 2
        pltpu.async_copy(rows_v.at[pl.ds(k * EGRP, EGRP)],
                         acc_sh.at[dst_v.at[slot, g % CHG]], sem_k, add=True)

    def _scat_wait(g, k, sem_k):
        slot = (g // CHG) % 2
        pltpu.make_async_copy(rows_v.at[pl.ds(k * EGRP, EGRP)],
                              acc_sh.at[dst_v.at[slot, g % CHG]], sem_k).wait()

    def _step(g, k, sems):
        @pl.when(g > 0)
        def _():
            _scat_wait(g - 1, (k + 4) % 5, sems[(k + 4) % 5])

        @pl.when(jnp.logical_and(g % CHG == 0, g + CHG < NG_E))
        def _():
            nslot = ((g // CHG) + 1) % 2
            pltpu.async_copy(
                srcr_hbm.at[wid, pl.ds((g // CHG + 1) * CHG, CHG)],
                src_v.at[nslot], sem_i)
            pltpu.async_copy(
                dstr_hbm.at[wid, pl.ds((g // CHG + 1) * CHG, CHG)],
                dst_v.at[nslot], sem_i)

        @pl.when(jnp.logical_and(g % CHG == CHG - 4, g + 4 < NG_E))
        def _():
            nslot = ((g // CHG) + 1) % 2
            pltpu.make_async_copy(
                srcr_hbm.at[wid, pl.ds((g // CHG + 1) * CHG, CHG)],
                src_v.at[nslot], sem_i).wait()
            pltpu.make_async_copy(
                dstr_hbm.at[wid, pl.ds((g // CHG + 1) * CHG, CHG)],
                dst_v.at[nslot], sem_i).wait()

        slot = (g // CHG) % 2
        pltpu.make_async_copy(
            g_hbm.at[src_v.at[slot, g % CHG]],
            rows_v.at[pl.ds(k * EGRP, EGRP)], sem).wait()

        @pl.when(g + 4 < NG_E)
        def _():
            _gather(g + 4)

        _scat_start(g, k, sems[k])

    def body(i, carry):
        sems = (sem_s0, sem_s1, sem_s2, sem_s3, sem_s4)
        for k in range(5):
            _step(5 * i + k, k, sems)
        return carry

    lax.fori_loop(0, NG_E // 5, body, 0)
    _scat_wait(NG_E - 1, (NG_E - 1) % 5, sem_s4)
    plsc.subcore_barrier()
    pltpu.sync_copy(acc_sh.at[pl.ds(s * rows_per_tile, rows_per_tile)],
                    out_hbm.at[c, pl.ds(s * rows_per_tile, rows_per_tile)])


_gcn_call = pl.kernel(
    _gcn_body,
    out_type=jax.ShapeDtypeStruct((2, N_PAD, HID), jnp.float32),
    mesh=_MESH,
    scratch_types=[
        pltpu.VMEM((2, CHG, EGRP), jnp.int32),
        pltpu.VMEM((2, CHG, EGRP), jnp.int32),
        pltpu.VMEM((5 * EGRP, HID), jnp.float32),
        pltpu.VMEM_SHARED((N_PAD, HID), jnp.float32),
        pltpu.SemaphoreType.DMA,
        pltpu.SemaphoreType.DMA,
        pltpu.SemaphoreType.DMA,
        pltpu.SemaphoreType.DMA,
        pltpu.SemaphoreType.DMA,
        pltpu.SemaphoreType.DMA,
        pltpu.SemaphoreType.DMA,
    ],
)


def _mm_body(a_ref, w_ref, o_ref):
    o_ref[...] = jnp.dot(a_ref[...], w_ref[...],
                         preferred_element_type=jnp.float32)


def _mm_fused_body(p0_ref, p1_ref, w_ref, o_ref):
    h = jnp.maximum(p0_ref[0] + p1_ref[0], 0.0)
    o_ref[...] = jnp.dot(h, w_ref[...], preferred_element_type=jnp.float32)


_MM_BLK = N_PAD // 2


def _mm(h, w):
    return pl.pallas_call(
        _mm_body,
        grid=(4,),
        in_specs=[
            pl.BlockSpec((_MM_BLK, HID), lambda i: (i, 0)),
            pl.BlockSpec((HID, HID), lambda i: (0, 0)),
        ],
        out_specs=pl.BlockSpec((_MM_BLK, HID), lambda i: (i, 0)),
        out_shape=jax.ShapeDtypeStruct((N_PAD, HID), jnp.float32),
    )(h, w)


def _mm_fused(p, w):
    return pl.pallas_call(
        _mm_fused_body,
        grid=(2,),
        in_specs=[
            pl.BlockSpec((1, _MM_BLK, HID), lambda i: (0, i, 0)),
            pl.BlockSpec((1, _MM_BLK, HID), lambda i: (1, i, 0)),
            pl.BlockSpec((HID, HID), lambda i: (0, 0)),
        ],
        out_specs=pl.BlockSpec((_MM_BLK, HID), lambda i: (i, 0)),
        out_shape=jax.ShapeDtypeStruct((N_PAD, HID), jnp.float32),
    )(p, p, w)


def kernel(x, edge_index, emb, W0, W1, W2, W3, W4, lin_w):
    xp = jnp.zeros((N_PAD, BAG), jnp.int32).at[:N].set(x)

    src = edge_index[0]
    dst = edge_index[1]
    pad_ids = (jnp.arange(E_PAD - E, dtype=jnp.int32) % (N_PAD - N)) + N
    srcp = jnp.concatenate([src, pad_ids]).reshape(NW, NG_E, EGRP)
    dstp = jnp.concatenate([dst, pad_ids]).reshape(NW, NG_E, EGRP)

    g = _emb_call(xp, emb, W0)
    for W in (W1, W2, W3, W4):
        p = _gcn_call(g, srcp, dstp)
        g = _mm_fused(p, W)
    p = _gcn_call(g, srcp, dstp)

    lin_pad = jnp.zeros((HID, 128), jnp.float32).at[:, :OUT].set(lin_w)
    out = _mm_fused(p, lin_pad)
    return out[:N, :OUT]


# speedup vs baseline: 1.2229x; 1.0009x over previous
import jax
import jax.numpy as jnp
from jax import lax
from jax.experimental import pallas as pl
from jax.experimental.pallas import tpu as pltpu
from jax.experimental.pallas import tpu_sc as plsc

N = 10000
E = 320000
VOCAB = 512
BAG = 16
HID = 128
OUT = 7

NW = 32
N_PAD = 10240
EGRP = 64
CHG = 16
NCH = 10
NG_E = NCH * CHG
EPW = NG_E * EGRP
E_PAD = NW * EPW

_MESH = plsc.VectorSubcoreMesh(core_axis_name="c", subcore_axis_name="s")


def _emb_body(x_ref, emb_ref, w_ref, o_ref):
    blk = x_ref.shape[0]
    cnt = jnp.zeros((blk, VOCAB), jnp.float32)
    iota = lax.broadcasted_iota(jnp.int32, (blk, VOCAB), 1)
    for b in range(BAG):
        cnt = cnt + (x_ref[:, b][:, None] == iota).astype(jnp.float32)
    h = jnp.maximum(
        jnp.dot(cnt, emb_ref[...], preferred_element_type=jnp.float32)
        * (1.0 / BAG), 0.0)
    o_ref[...] = jnp.dot(h, w_ref[...], preferred_element_type=jnp.float32)


def _emb_call(x, emb, w0):
    blk = N_PAD // 4
    return pl.pallas_call(
        _emb_body,
        grid=(4,),
        in_specs=[
            pl.BlockSpec((blk, BAG), lambda i: (i, 0)),
            pl.BlockSpec((VOCAB, HID), lambda i: (0, 0)),
            pl.BlockSpec((HID, HID), lambda i: (0, 0)),
        ],
        out_specs=pl.BlockSpec((blk, HID), lambda i: (i, 0)),
        out_shape=jax.ShapeDtypeStruct((N_PAD, HID), jnp.float32),
    )(x, emb, w0)


def _gcn_body(g_hbm, srcr_hbm, dstr_hbm, out_hbm, src_v, dst_v, rows_v,
              acc_sh, sem, sem_i, sem_s0, sem_s1, sem_s2, sem_s3, sem_s4):
    c = lax.axis_index("c")
    s = lax.axis_index("s")
    wid = s * 2 + c
    rows_per_tile = N_PAD // 16

    pltpu.sync_copy(srcr_hbm.at[wid, pl.ds(0, CHG)], src_v.at[0])
    pltpu.sync_copy(dstr_hbm.at[wid, pl.ds(0, CHG)], dst_v.at[0])

    def _gather(g):
        slot = (g // CHG) % 2
        pltpu.async_copy(g_hbm.at[src_v.at[slot, g % CHG]],
                         rows_v.at[pl.ds((g % 5) * EGRP, EGRP)], sem)

    for _pg in range(4):
        _gather(_pg)

    for r in range(EGRP):
        for cg in range(HID // 16):
            rows_v[4 * EGRP + r, pl.ds(cg * 16, 16)] = jnp.zeros(
                (16,), jnp.float32)
    for i in range(rows_per_tile // EGRP):
        pltpu.sync_copy(
            rows_v.at[pl.ds(4 * EGRP, EGRP)],
            acc_sh.at[pl.ds(s * rows_per_tile + i * EGRP, EGRP)])
    plsc.subcore_barrier()

    def _scat_start(g, k, sem_k):
        slot = (g // CHG) % 2
        pltpu.async_copy(rows_v.at[pl.ds(k * EGRP, EGRP)],
                         acc_sh.at[dst_v.at[slot, g % CHG]], sem_k, add=True)

    def _scat_wait(g, k, sem_k):
        slot = (g // CHG) % 2
        pltpu.make_async_copy(rows_v.at[pl.ds(k * EGRP, EGRP)],
                              acc_sh.at[dst_v.at[slot, g % CHG]], sem_k).wait()

    def _step(g, k, sems):
        @pl.when(g > 0)
        def _():
            _scat_wait(g - 1, (k + 4) % 5, sems[(k + 4) % 5])

        @pl.when(jnp.logical_and(g % CHG == 0, g + CHG < NG_E))
        def _():
            nslot = ((g // CHG) + 1) % 2
            pltpu.async_copy(
                srcr_hbm.at[wid, pl.ds((g // CHG + 1) * CHG, CHG)],
                src_v.at[nslot], sem_i)
            pltpu.async_copy(
                dstr_hbm.at[wid, pl.ds((g // CHG + 1) * CHG, CHG)],
                dst_v.at[nslot], sem_i)

        @pl.when(jnp.logical_and(g % CHG == CHG - 4, g + 4 < NG_E))
        def _():
            nslot = ((g // CHG) + 1) % 2
            pltpu.make_async_copy(
                srcr_hbm.at[wid, pl.ds((g // CHG + 1) * CHG, CHG)],
                src_v.at[nslot], sem_i).wait()
            pltpu.make_async_copy(
                dstr_hbm.at[wid, pl.ds((g // CHG + 1) * CHG, CHG)],
                dst_v.at[nslot], sem_i).wait()

        slot = (g // CHG) % 2
        pltpu.make_async_copy(
            g_hbm.at[src_v.at[slot, g % CHG]],
            rows_v.at[pl.ds(k * EGRP, EGRP)], sem).wait()

        @pl.when(g + 4 < NG_E)
        def _():
            _gather(g + 4)

        _scat_start(g, k, sems[k])

    def body(i, carry):
        sems = (sem_s0, sem_s1, sem_s2, sem_s3, sem_s4)
        for k in range(5):
            _step(5 * i + k, k, sems)
        return carry

    lax.fori_loop(0, NG_E // 5, body, 0)
    _scat_wait(NG_E - 1, (NG_E - 1) % 5, sem_s4)
    plsc.subcore_barrier()
    pltpu.sync_copy(acc_sh.at[pl.ds(s * rows_per_tile, rows_per_tile)],
                    out_hbm.at[c, pl.ds(s * rows_per_tile, rows_per_tile)])


_gcn_call = pl.kernel(
    _gcn_body,
    out_type=jax.ShapeDtypeStruct((2, N_PAD, HID), jnp.float32),
    mesh=_MESH,
    scratch_types=[
        pltpu.VMEM((2, CHG, EGRP), jnp.int32),
        pltpu.VMEM((2, CHG, EGRP), jnp.int32),
        pltpu.VMEM((5 * EGRP, HID), jnp.float32),
        pltpu.VMEM_SHARED((N_PAD, HID), jnp.float32),
        pltpu.SemaphoreType.DMA,
        pltpu.SemaphoreType.DMA,
        pltpu.SemaphoreType.DMA,
        pltpu.SemaphoreType.DMA,
        pltpu.SemaphoreType.DMA,
        pltpu.SemaphoreType.DMA,
        pltpu.SemaphoreType.DMA,
    ],
)


def _mm_fused_body(p0_ref, p1_ref, w_ref, o_ref):
    h = jnp.maximum(p0_ref[0] + p1_ref[0], 0.0)
    o_ref[...] = jnp.dot(h, w_ref[...], preferred_element_type=jnp.float32)


_MM_BLK = N_PAD // 2


def _mm_fused(p, w):
    return pl.pallas_call(
        _mm_fused_body,
        grid=(2,),
        in_specs=[
            pl.BlockSpec((1, _MM_BLK, HID), lambda i: (0, i, 0)),
            pl.BlockSpec((1, _MM_BLK, HID), lambda i: (1, i, 0)),
            pl.BlockSpec((HID, HID), lambda i: (0, 0)),
        ],
        out_specs=pl.BlockSpec((_MM_BLK, HID), lambda i: (i, 0)),
        out_shape=jax.ShapeDtypeStruct((N_PAD, HID), jnp.float32),
    )(p, p, w)


def kernel(x, edge_index, emb, W0, W1, W2, W3, W4, lin_w):
    xp = jnp.zeros((N_PAD, BAG), jnp.int32).at[:N].set(x)

    src = edge_index[0]
    dst = edge_index[1]
    pad_ids = (jnp.arange(E_PAD - E, dtype=jnp.int32) % (N_PAD - N)) + N
    srcp = jnp.concatenate([src, pad_ids]).reshape(NW, NG_E, EGRP)
    dstp = jnp.concatenate([dst, pad_ids]).reshape(NW, NG_E, EGRP)

    g = _emb_call(xp, emb, W0)
    for W in (W1, W2, W3, W4):
        p = _gcn_call(g, srcp, dstp)
        g = _mm_fused(p, W)
    p = _gcn_call(g, srcp, dstp)

    lin_pad = jnp.zeros((HID, 128), jnp.float32).at[:, :OUT].set(lin_w)
    out = _mm_fused(p, lin_pad)
    return out[:N, :OUT]
